# baseline (speedup 1.0000x reference)
import jax
import jax.numpy as jnp
from jax.experimental import pallas as pl
from jax.experimental.pallas import tpu as pltpu

_NUM_CLASSES = 30
_IMG_LO = 1.0
_IMG_HI = 799.0


def _body(pt_ref, gt_ref, ph_ref, lab_ref, x_ref, cn_ref, mt_ref, pf_ref,
          mask_ref):
    i = pl.program_id(0)

    @pl.when(i == 0)
    def _mask_phase():
        n = pt_ref.shape[1]
        px1 = jnp.clip(pt_ref[0:1, :], _IMG_LO, _IMG_HI)
        py1 = jnp.clip(pt_ref[1:2, :], _IMG_LO, _IMG_HI)
        px2 = jnp.clip(pt_ref[2:3, :], _IMG_LO, _IMG_HI)
        py2 = jnp.clip(pt_ref[3:4, :], _IMG_LO, _IMG_HI)
        area_b = (px2 - px1 + 1.0) * (py2 - py1 + 1.0)

        iou_rows = []
        for g in range(8):
            gx1 = jnp.clip(gt_ref[g, 0], _IMG_LO, _IMG_HI)
            gy1 = jnp.clip(gt_ref[g, 1], _IMG_LO, _IMG_HI)
            gx2 = jnp.clip(gt_ref[g, 2], _IMG_LO, _IMG_HI)
            gy2 = jnp.clip(gt_ref[g, 3], _IMG_LO, _IMG_HI)
            iw = jnp.maximum(jnp.minimum(px2, gx2) - jnp.maximum(px1, gx1)
                             + 1.0, 0.0)
            ih = jnp.maximum(jnp.minimum(py2, gy2) - jnp.maximum(py1, gy1)
                             + 1.0, 0.0)
            inter = iw * ih
            area_g = (gx2 - gx1 + 1.0) * (gy2 - gy1 + 1.0)
            iou_rows.append(inter / (area_b + area_g - inter))
        iou = jnp.concatenate(iou_rows, axis=0)

        cls_iota = jax.lax.broadcasted_iota(jnp.int32, (32, 1), 0)
        cn = jnp.zeros((32, n), jnp.float32)
        for g in range(8):
            onehot = (cls_iota == lab_ref[g]).astype(jnp.float32)
            cn = jnp.maximum(cn, onehot * iou_rows[g])
        cn_ref[...] = cn

        mrows = []
        for g in range(8):
            acc = iou_rows[g]
            for g2 in range(8):
                if g2 == g:
                    continue
                same = lab_ref[g] == lab_ref[g2]
                acc = jnp.maximum(acc, jnp.where(same, iou_rows[g2], 0.0))
            mrows.append((acc > 0.6).astype(jnp.float32))
        mask = jnp.concatenate(mrows, axis=0)
        mask_ref[...] = mask

        src_w = px2 - px1
        src_h = py2 - py1
        src_cx = px1 + 0.5 * src_w
        src_cy = py1 + 0.5 * src_h
        rows = []
        for g in range(8):
            hx1 = jnp.clip(ph_ref[g, 0], _IMG_LO, _IMG_HI)
            hy1 = jnp.clip(ph_ref[g, 1], _IMG_LO, _IMG_HI)
            hx2 = jnp.clip(ph_ref[g, 2], _IMG_LO, _IMG_HI)
            hy2 = jnp.clip(ph_ref[g, 3], _IMG_LO, _IMG_HI)
            gw = hx2 - hx1
            gh = hy2 - hy1
            gcx = hx1 + 0.5 * gw
            gcy = hy1 + 0.5 * gh
            m = mrows[g]
            rows.append(((gcx - src_cx) / src_w) * m)
            rows.append(((gcy - src_cy) / src_h) * m)
            rows.append(jnp.log(gw / src_w) * m)
            rows.append(jnp.log(gh / src_h) * m)
        mt_ref[...] = jnp.concatenate(rows, axis=0)

    pf_ref[...] = jnp.dot(mask_ref[...], x_ref[...],
                          preferred_element_type=jnp.float32)


def kernel(x, proposals, gt_bbox, gt_labels):
    n, d = x.shape
    g = gt_bbox.shape[0]
    dblk = 256
    pt = proposals.T
    ph = proposals[:g]
    labs = gt_labels.astype(jnp.int32)

    cn, mt, pf = pl.pallas_call(
        _body,
        grid=(d // dblk,),
        in_specs=[
            pl.BlockSpec((4, n), lambda i: (0, 0)),
            pl.BlockSpec(memory_space=pltpu.SMEM),
            pl.BlockSpec(memory_space=pltpu.SMEM),
            pl.BlockSpec(memory_space=pltpu.SMEM),
            pl.BlockSpec((n, dblk), lambda i: (0, i)),
        ],
        out_specs=[
            pl.BlockSpec((32, n), lambda i: (0, 0)),
            pl.BlockSpec((32, n), lambda i: (0, 0)),
            pl.BlockSpec((g, dblk), lambda i: (0, i)),
        ],
        out_shape=[
            jax.ShapeDtypeStruct((32, n), jnp.float32),
            jax.ShapeDtypeStruct((32, n), jnp.float32),
            jax.ShapeDtypeStruct((g, d), jnp.float32),
        ],
        scratch_shapes=[pltpu.VMEM((8, n), jnp.float32)],
    )(pt, gt_bbox, ph, labs, x)

    overlap = cn[:_NUM_CLASSES].T
    masked_targets = mt.reshape(g, 4, n).transpose(0, 2, 1)
    return overlap, masked_targets, pf
